# baseline (speedup 1.0000x reference)
import functools

import jax
import jax.numpy as jnp
from jax import lax
from jax.experimental import pallas as pl
from jax.experimental.pallas import tpu as pltpu
from jax.experimental.pallas import tpu_sc as plsc

N = 10000
F_IN = 128
H = 64
C = 40
E = 320000

NC = 2
NS = 16
NW = NC * NS
CHUNK = 128
CPW = 80
EPW = CPW * CHUNK
EPAD = NW * EPW
NPAD = 10240
RPT = NPAD // NS
NPK = NPAD // 2
NBUF = 3

_mesh = plsc.VectorSubcoreMesh(
    core_axis_name="c", subcore_axis_name="s", num_cores=NC, num_subcores=NS)
_sc_params = pltpu.CompilerParams(use_tc_tiling_on_sc=False,
                                  needs_layout_passes=False)


@functools.partial(
    pl.kernel,
    out_type=jax.ShapeDtypeStruct((NC, NPAD // 2, 128), jnp.float32),
    mesh=_mesh,
    compiler_params=_sc_params,
    scratch_types=[
        pltpu.VMEM((CPW, CHUNK), jnp.int32),
        pltpu.VMEM((CHUNK,), jnp.float32),
        pltpu.VMEM((CHUNK,), jnp.float32),
        pltpu.VMEM((RPT,), jnp.float32),
        pltpu.VMEM((RPT // 2, 128), jnp.float32),
        pltpu.VMEM_SHARED((NPAD,), jnp.float32),
    ],
)
def _deg_kernel(c3_hbm, out_hbm, cidx_v, ones_v, zero_v, dv_v, pk_v, dacc_sh):
    cid = lax.axis_index("c")
    sid = lax.axis_index("s")
    wid = cid * NS + sid

    pltpu.sync_copy(c3_hbm.at[wid], cidx_v)

    def fill(i, _):
        ones_v[pl.ds(i * 16, 16)] = jnp.ones((16,), jnp.float32)
        zero_v[pl.ds(i * 16, 16)] = jnp.zeros((16,), jnp.float32)
        return 0

    lax.fori_loop(0, CHUNK // 16, fill, 0)

    base = sid * RPT
    for k in range(RPT // CHUNK):
        pltpu.sync_copy(zero_v, dacc_sh.at[pl.ds(base + k * CHUNK, CHUNK)])
    plsc.subcore_barrier()

    def body(j, _):
        pltpu.sync_copy(ones_v, dacc_sh.at[cidx_v.at[j]], add=True)
        return 0

    lax.fori_loop(0, CPW, body, 0)

    plsc.subcore_barrier()
    pltpu.sync_copy(dacc_sh.at[pl.ds(base, RPT)], dv_v)

    def expand(m, _):
        row = plsc.load_gather(dv_v, [jnp.full((16,), m, jnp.int32)])
        for q in range(4):
            pk_v[m // 2, pl.ds((m % 2) * H + q * 16, 16)] = row
        return 0

    lax.fori_loop(0, RPT, expand, 0)
    pltpu.sync_copy(pk_v, out_hbm.at[cid, pl.ds(sid * (RPT // 2), RPT // 2)])


D = H
_LW = D // 16


@functools.partial(
    pl.kernel,
    out_type=jax.ShapeDtypeStruct((NC, NPAD, D), jnp.float32),
    mesh=_mesh,
    compiler_params=_sc_params,
    scratch_types=[
        pltpu.VMEM((CPW, CHUNK), jnp.int32),
        pltpu.VMEM((CPW, CHUNK), jnp.int32),
        [pltpu.VMEM((CHUNK, D), jnp.float32)] * NBUF,
        [pltpu.SemaphoreType.DMA] * NBUF,
        [pltpu.SemaphoreType.DMA] * NBUF,
        pltpu.VMEM_SHARED((NPAD, D), jnp.float32),
        pltpu.VMEM_SHARED((NPAD, D), jnp.float32),
    ],
)
def _prop(r3_hbm, c3_hbm, hs_hbm, out_hbm, ridx_v, cidx_v, bufs, gsems,
          ssems, acc_sh, table_sh):
    cid = lax.axis_index("c")
    sid = lax.axis_index("s")
    wid = cid * NS + sid

    pltpu.sync_copy(r3_hbm.at[wid], ridx_v)
    pltpu.sync_copy(c3_hbm.at[wid], cidx_v)

    def zfill(i, _):
        bufs[0][i // _LW, pl.ds((i % _LW) * 16, 16)] = jnp.zeros(
            (16,), jnp.float32)
        return 0

    lax.fori_loop(0, CHUNK * _LW, zfill, 0)

    base = sid * RPT
    for k in range(RPT // CHUNK):
        pltpu.sync_copy(bufs[0], acc_sh.at[pl.ds(base + k * CHUNK, CHUNK)])
    pltpu.sync_copy(hs_hbm.at[pl.ds(base, RPT)],
                    table_sh.at[pl.ds(base, RPT)])
    plsc.subcore_barrier()

    def cb_fire(j, b):
        @pl.when(j < CPW)
        def _():
            pltpu.async_copy(table_sh.at[ridx_v.at[j]], bufs[b], gsems[b])

    def cb_pair(ja, jb_, na, nb):
        for (j, b, nxt) in ((ja, 0, na), (jb_, 1, nb)):
            pltpu.make_async_copy(table_sh.at[ridx_v.at[j]], bufs[b],
                                  gsems[b]).wait()
            pltpu.async_copy(bufs[b], acc_sh.at[cidx_v.at[j]], ssems[b],
                             add=True)
        for (j, b, nxt) in ((ja, 0, na), (jb_, 1, nb)):
            pltpu.make_async_copy(bufs[b], acc_sh.at[cidx_v.at[j]],
                                  ssems[b]).wait()
            cb_fire(nxt, b)

    def hbm_chunk(j):
        pltpu.make_async_copy(hs_hbm.at[ridx_v.at[j]], bufs[2],
                              gsems[2]).wait()
        pltpu.async_copy(bufs[2], acc_sh.at[cidx_v.at[j]], ssems[2],
                         add=True)
        pltpu.make_async_copy(bufs[2], acc_sh.at[cidx_v.at[j]],
                              ssems[2]).wait()

        @pl.when(j + 4 < CPW)
        def _():
            pltpu.async_copy(hs_hbm.at[ridx_v.at[j + 4]], bufs[2], gsems[2])

    pltpu.async_copy(table_sh.at[ridx_v.at[0]], bufs[0], gsems[0])
    pltpu.async_copy(table_sh.at[ridx_v.at[1]], bufs[1], gsems[1])
    pltpu.async_copy(hs_hbm.at[ridx_v.at[3]], bufs[2], gsems[2])

    def body(i, _):
        g = 8 * i
        cb_pair(g + 0, g + 1, g + 2, g + 4)
        hbm_chunk(g + 3)
        cb_pair(g + 2, g + 4, g + 5, g + 6)
        hbm_chunk(g + 7)
        cb_pair(g + 5, g + 6, g + 8, g + 9)
        return 0

    lax.fori_loop(0, CPW // 8, body, 0)

    plsc.subcore_barrier()
    pltpu.sync_copy(acc_sh.at[pl.ds(base, RPT)],
                    out_hbm.at[cid, pl.ds(base, RPT)])


def _pack_mask():
    rowio = lax.broadcasted_iota(jnp.int32, (NPK, 128), 0)
    laneio = lax.broadcasted_iota(jnp.int32, (NPK, 128), 1)
    nodev = 2 * rowio + laneio // H
    return jnp.where(nodev < N, 1.0, 0.0).astype(jnp.float32)


def _tc1(x2_ref, w1x_ref, deg_ref, hs1_ref, dinv_ref):
    dpack = lax.rsqrt(deg_ref[0] + deg_ref[1] + 1.0)
    h_p = jnp.dot(x2_ref[...], w1x_ref[...],
                  preferred_element_type=jnp.float32)
    hs1_ref[...] = h_p * dpack
    dinv_ref[...] = dpack


def _tc2(acc_ref, hs1_ref, dinv_ref, g_ref, be_ref, b1_ref, w2x_ref,
         hs2_ref):
    dpack = dinv_ref[...]
    maskf = _pack_mask()
    s = ((acc_ref[0] + acc_ref[1] + hs1_ref[...]) * dpack
         + b1_ref[...]) * maskf
    inv_n = jnp.float32(1.0 / N)
    mu_p = jnp.sum(s, axis=0, keepdims=True) * inv_n
    mu64 = mu_p[:, :H] + mu_p[:, H:]
    mu = jnp.concatenate([mu64, mu64], axis=1)
    v_p = jnp.sum(s * s, axis=0, keepdims=True) * inv_n
    v64 = v_p[:, :H] + v_p[:, H:]
    var = jnp.concatenate([v64, v64], axis=1) - mu * mu
    g = g_ref[...] * (s - mu) * lax.rsqrt(var + 1e-5) + be_ref[...]
    g = jnp.maximum(g, 0.0) * maskf
    h2 = jnp.dot(g, w2x_ref[...], preferred_element_type=jnp.float32)
    hs2_ref[...] = h2 * dpack


def _tc3(acc_ref, hs2_ref, dinv_ref, b2_ref, out_ref):
    out_ref[...] = ((acc_ref[0] + acc_ref[1] + hs2_ref[...]) * dinv_ref[...]
                    + b2_ref[...])


def kernel(x, edge_index, W1, b1, gamma1, beta1, W2, b2):
    pad = jnp.full((EPAD - E,), N, dtype=jnp.int32)
    r_p = jnp.concatenate([edge_index[0], pad]).reshape(NW, CPW, CHUNK)
    c_p = jnp.concatenate([edge_index[1], pad]).reshape(NW, CPW, CHUNK)
    x2 = jnp.pad(x, ((0, NPAD - N), (0, 0))).reshape(NPK, 2 * F_IN)
    z = jnp.zeros((F_IN, H), jnp.float32)
    w1x = jnp.concatenate(
        [jnp.concatenate([W1, z], axis=1), jnp.concatenate([z, W1], axis=1)],
        axis=0)
    w2p = jnp.pad(W2, ((0, 0), (0, H - C)))
    z2 = jnp.zeros((H, H), jnp.float32)
    w2x = jnp.concatenate(
        [jnp.concatenate([w2p, z2], axis=1),
         jnp.concatenate([z2, w2p], axis=1)], axis=0)
    b1p = jnp.tile(b1, 2).reshape(1, 2 * H)
    b2p = jnp.tile(jnp.pad(b2, (0, H - C)), 2).reshape(1, 2 * H)
    g1p = jnp.tile(gamma1, 2).reshape(1, 2 * H)
    be1p = jnp.tile(beta1, 2).reshape(1, 2 * H)

    degp = _deg_kernel(c_p)

    hs1_p, dinv_p = pl.pallas_call(
        _tc1,
        out_shape=[jax.ShapeDtypeStruct((NPK, 128), jnp.float32),
                   jax.ShapeDtypeStruct((NPK, 128), jnp.float32)],
    )(x2, w1x, degp)

    acc1 = _prop(r_p, c_p, hs1_p.reshape(NPAD, D))
    acc1v = acc1.reshape(NC, NPK, 128)

    hs2_p = pl.pallas_call(
        _tc2,
        out_shape=jax.ShapeDtypeStruct((NPK, 128), jnp.float32),
    )(acc1v, hs1_p, dinv_p, g1p, be1p, b1p, w2x)

    acc2 = _prop(r_p, c_p, hs2_p.reshape(NPAD, D))
    acc2v = acc2.reshape(NC, NPK, 128)

    out_p = pl.pallas_call(
        _tc3,
        out_shape=jax.ShapeDtypeStruct((NPK, 128), jnp.float32),
    )(acc2v, hs2_p, dinv_p, b2p)

    return out_p.reshape(NPAD, H)[:N, :C]


# speedup vs baseline: 1.1410x; 1.1410x over previous
import functools

import jax
import jax.numpy as jnp
from jax import lax
from jax.experimental import pallas as pl
from jax.experimental.pallas import tpu as pltpu
from jax.experimental.pallas import tpu_sc as plsc

N = 10000
F_IN = 128
H = 64
C = 40
E = 320000

NC = 2
NS = 16
NW = NC * NS
CHUNK = 128
CPW = 80
EPW = CPW * CHUNK
EPAD = NW * EPW
NPAD = 10240
RPT = NPAD // NS
NPK = NPAD // 2
NBUF = 2

_mesh = plsc.VectorSubcoreMesh(
    core_axis_name="c", subcore_axis_name="s", num_cores=NC, num_subcores=NS)
_sc_params = pltpu.CompilerParams(use_tc_tiling_on_sc=False,
                                  needs_layout_passes=False)


@functools.partial(
    pl.kernel,
    out_type=jax.ShapeDtypeStruct((NC, NPAD // 2, 128), jnp.float32),
    mesh=_mesh,
    compiler_params=_sc_params,
    scratch_types=[
        pltpu.VMEM((CPW, CHUNK), jnp.int32),
        pltpu.VMEM((CHUNK,), jnp.float32),
        pltpu.VMEM((CHUNK,), jnp.float32),
        pltpu.VMEM((RPT,), jnp.float32),
        pltpu.VMEM((RPT // 2, 128), jnp.float32),
        pltpu.VMEM_SHARED((NPAD,), jnp.float32),
    ],
)
def _deg_kernel(c3_hbm, out_hbm, cidx_v, ones_v, zero_v, dv_v, pk_v, dacc_sh):
    cid = lax.axis_index("c")
    sid = lax.axis_index("s")
    wid = cid * NS + sid

    pltpu.sync_copy(c3_hbm.at[wid], cidx_v)

    def fill(i, _):
        ones_v[pl.ds(i * 16, 16)] = jnp.ones((16,), jnp.float32)
        zero_v[pl.ds(i * 16, 16)] = jnp.zeros((16,), jnp.float32)
        return 0

    lax.fori_loop(0, CHUNK // 16, fill, 0)

    base = sid * RPT
    for k in range(RPT // CHUNK):
        pltpu.sync_copy(zero_v, dacc_sh.at[pl.ds(base + k * CHUNK, CHUNK)])
    plsc.subcore_barrier()

    def body(j, _):
        pltpu.sync_copy(ones_v, dacc_sh.at[cidx_v.at[j]], add=True)
        return 0

    lax.fori_loop(0, CPW, body, 0)

    plsc.subcore_barrier()
    pltpu.sync_copy(dacc_sh.at[pl.ds(base, RPT)], dv_v)

    def expand(m, _):
        row = plsc.load_gather(dv_v, [jnp.full((16,), m, jnp.int32)])
        for q in range(4):
            pk_v[m // 2, pl.ds((m % 2) * H + q * 16, 16)] = row
        return 0

    lax.fori_loop(0, RPT, expand, 0)
    pltpu.sync_copy(pk_v, out_hbm.at[cid, pl.ds(sid * (RPT // 2), RPT // 2)])


D = H
_LW = D // 16


@functools.partial(
    pl.kernel,
    out_type=jax.ShapeDtypeStruct((NC, NPAD, D), jnp.float32),
    mesh=_mesh,
    compiler_params=_sc_params,
    scratch_types=[
        pltpu.VMEM((CPW, CHUNK), jnp.int32),
        pltpu.VMEM((CPW, CHUNK), jnp.int32),
        [pltpu.VMEM((CHUNK, D), jnp.float32)] * NBUF,
        [pltpu.SemaphoreType.DMA] * NBUF,
        [pltpu.SemaphoreType.DMA] * NBUF,
        pltpu.VMEM_SHARED((NPAD, D), jnp.float32),
        pltpu.VMEM_SHARED((NPAD, D), jnp.float32),
    ],
)
def _prop(r3_hbm, c3_hbm, hs_hbm, out_hbm, ridx_v, cidx_v, bufs, gsems,
          ssems, acc_sh, table_sh):
    cid = lax.axis_index("c")
    sid = lax.axis_index("s")
    wid = cid * NS + sid

    pltpu.sync_copy(r3_hbm.at[wid], ridx_v)
    pltpu.sync_copy(c3_hbm.at[wid], cidx_v)

    def zfill(i, _):
        bufs[0][i // _LW, pl.ds((i % _LW) * 16, 16)] = jnp.zeros(
            (16,), jnp.float32)
        return 0

    lax.fori_loop(0, CHUNK * _LW, zfill, 0)

    base = sid * RPT
    for k in range(RPT // CHUNK):
        pltpu.sync_copy(bufs[0], acc_sh.at[pl.ds(base + k * CHUNK, CHUNK)])
    pltpu.sync_copy(hs_hbm.at[pl.ds(base, RPT)],
                    table_sh.at[pl.ds(base, RPT)])
    plsc.subcore_barrier()

    for k in range(NBUF):
        pltpu.async_copy(table_sh.at[ridx_v.at[k]], bufs[k], gsems[k])

    def body(i, _):
        jb = NBUF * i
        for k in range(NBUF):
            j = jb + k
            pltpu.make_async_copy(table_sh.at[ridx_v.at[j]], bufs[k],
                                  gsems[k]).wait()
            pltpu.async_copy(bufs[k], acc_sh.at[cidx_v.at[j]], ssems[k],
                             add=True)
        for k in range(NBUF):
            j = jb + k
            pltpu.make_async_copy(bufs[k], acc_sh.at[cidx_v.at[j]],
                                  ssems[k]).wait()

            @pl.when(j + NBUF < CPW)
            def _():
                pltpu.async_copy(table_sh.at[ridx_v.at[j + NBUF]], bufs[k],
                                 gsems[k])
        return 0

    lax.fori_loop(0, CPW // NBUF, body, 0)

    plsc.subcore_barrier()
    pltpu.sync_copy(acc_sh.at[pl.ds(base, RPT)],
                    out_hbm.at[cid, pl.ds(base, RPT)])


def _pack_mask():
    rowio = lax.broadcasted_iota(jnp.int32, (NPK, 128), 0)
    laneio = lax.broadcasted_iota(jnp.int32, (NPK, 128), 1)
    nodev = 2 * rowio + laneio // H
    return jnp.where(nodev < N, 1.0, 0.0).astype(jnp.float32)


def _tc1(x2_ref, w1x_ref, deg_ref, hs1_ref, dinv_ref):
    dpack = lax.rsqrt(deg_ref[0] + deg_ref[1] + 1.0)
    h_p = jnp.dot(x2_ref[...], w1x_ref[...],
                  preferred_element_type=jnp.float32)
    hs1_ref[...] = h_p * dpack
    dinv_ref[...] = dpack


def _tc2(acc_ref, hs1_ref, dinv_ref, g_ref, be_ref, b1_ref, w2x_ref,
         hs2_ref):
    dpack = dinv_ref[...]
    maskf = _pack_mask()
    s = ((acc_ref[0] + acc_ref[1] + hs1_ref[...]) * dpack
         + b1_ref[...]) * maskf
    inv_n = jnp.float32(1.0 / N)
    mu_p = jnp.sum(s, axis=0, keepdims=True) * inv_n
    mu64 = mu_p[:, :H] + mu_p[:, H:]
    mu = jnp.concatenate([mu64, mu64], axis=1)
    v_p = jnp.sum(s * s, axis=0, keepdims=True) * inv_n
    v64 = v_p[:, :H] + v_p[:, H:]
    var = jnp.concatenate([v64, v64], axis=1) - mu * mu
    g = g_ref[...] * (s - mu) * lax.rsqrt(var + 1e-5) + be_ref[...]
    g = jnp.maximum(g, 0.0) * maskf
    h2 = jnp.dot(g, w2x_ref[...], preferred_element_type=jnp.float32)
    hs2_ref[...] = h2 * dpack


def _tc3(acc_ref, hs2_ref, dinv_ref, b2_ref, out_ref):
    out_ref[...] = ((acc_ref[0] + acc_ref[1] + hs2_ref[...]) * dinv_ref[...]
                    + b2_ref[...])


def kernel(x, edge_index, W1, b1, gamma1, beta1, W2, b2):
    pad = jnp.full((EPAD - E,), N, dtype=jnp.int32)
    r_p = jnp.concatenate([edge_index[0], pad]).reshape(NW, CPW, CHUNK)
    c_p = jnp.concatenate([edge_index[1], pad]).reshape(NW, CPW, CHUNK)
    x2 = jnp.pad(x, ((0, NPAD - N), (0, 0))).reshape(NPK, 2 * F_IN)
    z = jnp.zeros((F_IN, H), jnp.float32)
    w1x = jnp.concatenate(
        [jnp.concatenate([W1, z], axis=1), jnp.concatenate([z, W1], axis=1)],
        axis=0)
    w2p = jnp.pad(W2, ((0, 0), (0, H - C)))
    z2 = jnp.zeros((H, H), jnp.float32)
    w2x = jnp.concatenate(
        [jnp.concatenate([w2p, z2], axis=1),
         jnp.concatenate([z2, w2p], axis=1)], axis=0)
    b1p = jnp.tile(b1, 2).reshape(1, 2 * H)
    b2p = jnp.tile(jnp.pad(b2, (0, H - C)), 2).reshape(1, 2 * H)
    g1p = jnp.tile(gamma1, 2).reshape(1, 2 * H)
    be1p = jnp.tile(beta1, 2).reshape(1, 2 * H)

    degp = _deg_kernel(c_p)

    hs1_p, dinv_p = pl.pallas_call(
        _tc1,
        out_shape=[jax.ShapeDtypeStruct((NPK, 128), jnp.float32),
                   jax.ShapeDtypeStruct((NPK, 128), jnp.float32)],
    )(x2, w1x, degp)

    acc1 = _prop(r_p, c_p, hs1_p.reshape(NPAD, D))
    acc1v = acc1.reshape(NC, NPK, 128)

    hs2_p = pl.pallas_call(
        _tc2,
        out_shape=jax.ShapeDtypeStruct((NPK, 128), jnp.float32),
    )(acc1v, hs1_p, dinv_p, g1p, be1p, b1p, w2x)

    acc2 = _prop(r_p, c_p, hs2_p.reshape(NPAD, D))
    acc2v = acc2.reshape(NC, NPK, 128)

    out_p = pl.pallas_call(
        _tc3,
        out_shape=jax.ShapeDtypeStruct((NPK, 128), jnp.float32),
    )(acc2v, hs2_p, dinv_p, b2p)

    return out_p.reshape(NPAD, H)[:N, :C]


# speedup vs baseline: 1.1583x; 1.0152x over previous
import functools

import jax
import jax.numpy as jnp
from jax import lax
from jax.experimental import pallas as pl
from jax.experimental.pallas import tpu as pltpu
from jax.experimental.pallas import tpu_sc as plsc

N = 10000
F_IN = 128
H = 64
C = 40
E = 320000

NC = 2
NS = 16
NW = NC * NS
CHUNK = 128
CPW = 80
EPW = CPW * CHUNK
EPAD = NW * EPW
NPAD = 10240
RPT = NPAD // NS
NPK = NPAD // 2
NBUF = 2

_mesh = plsc.VectorSubcoreMesh(
    core_axis_name="c", subcore_axis_name="s", num_cores=NC, num_subcores=NS)
_sc_params = pltpu.CompilerParams(use_tc_tiling_on_sc=False,
                                  needs_layout_passes=False)


@functools.partial(
    pl.kernel,
    out_type=jax.ShapeDtypeStruct((NC, NPAD // 2, 128), jnp.float32),
    mesh=_mesh,
    compiler_params=_sc_params,
    scratch_types=[
        pltpu.VMEM((CPW, CHUNK), jnp.int32),
        pltpu.VMEM((CHUNK,), jnp.float32),
        pltpu.VMEM((CHUNK,), jnp.float32),
        pltpu.VMEM((RPT,), jnp.float32),
        pltpu.VMEM((RPT // 2, 128), jnp.float32),
        pltpu.VMEM_SHARED((NPAD,), jnp.float32),
    ],
)
def _deg_kernel(c3_hbm, out_hbm, cidx_v, ones_v, zero_v, dv_v, pk_v, dacc_sh):
    cid = lax.axis_index("c")
    sid = lax.axis_index("s")
    wid = cid * NS + sid

    pltpu.sync_copy(c3_hbm.at[wid], cidx_v)

    def fill(i, _):
        ones_v[pl.ds(i * 16, 16)] = jnp.ones((16,), jnp.float32)
        zero_v[pl.ds(i * 16, 16)] = jnp.zeros((16,), jnp.float32)
        return 0

    lax.fori_loop(0, CHUNK // 16, fill, 0)

    base = sid * RPT
    for k in range(RPT // CHUNK):
        pltpu.sync_copy(zero_v, dacc_sh.at[pl.ds(base + k * CHUNK, CHUNK)])
    plsc.subcore_barrier()

    def body(j, _):
        pltpu.sync_copy(ones_v, dacc_sh.at[cidx_v.at[j]], add=True)
        return 0

    lax.fori_loop(0, CPW, body, 0)

    plsc.subcore_barrier()
    pltpu.sync_copy(dacc_sh.at[pl.ds(base, RPT)], dv_v)

    def expand(m, _):
        row = plsc.load_gather(dv_v, [jnp.full((16,), m, jnp.int32)])
        for q in range(4):
            pk_v[m // 2, pl.ds((m % 2) * H + q * 16, 16)] = row
        return 0

    lax.fori_loop(0, RPT, expand, 0)
    pltpu.sync_copy(pk_v, out_hbm.at[cid, pl.ds(sid * (RPT // 2), RPT // 2)])


D = H
_LW = D // 16


@functools.partial(
    pl.kernel,
    out_type=jax.ShapeDtypeStruct((NC, NPAD, D), jnp.float32),
    mesh=_mesh,
    compiler_params=_sc_params,
    scratch_types=[
        pltpu.VMEM((CPW, CHUNK), jnp.int32),
        pltpu.VMEM((CPW, CHUNK), jnp.int32),
        [pltpu.VMEM((CHUNK, D), jnp.float32)] * NBUF,
        [pltpu.SemaphoreType.DMA] * NBUF,
        [pltpu.SemaphoreType.DMA] * NBUF,
        pltpu.VMEM_SHARED((NPAD, D), jnp.float32),
        pltpu.VMEM_SHARED((NPAD, D), jnp.float32),
    ],
)
def _prop(r3_hbm, c3_hbm, hs_hbm, out_hbm, ridx_v, cidx_v, bufs, gsems,
          ssems, acc_sh, table_sh):
    cid = lax.axis_index("c")
    sid = lax.axis_index("s")
    wid = cid * NS + sid

    pltpu.sync_copy(r3_hbm.at[wid], ridx_v)
    pltpu.sync_copy(c3_hbm.at[wid], cidx_v)

    def zfill(i, _):
        bufs[0][i // _LW, pl.ds((i % _LW) * 16, 16)] = jnp.zeros(
            (16,), jnp.float32)
        return 0

    lax.fori_loop(0, CHUNK * _LW, zfill, 0)

    base = sid * RPT
    for k in range(RPT // CHUNK):
        pltpu.sync_copy(bufs[0], acc_sh.at[pl.ds(base + k * CHUNK, CHUNK)])
    pltpu.sync_copy(hs_hbm.at[pl.ds(base, RPT)],
                    table_sh.at[pl.ds(base, RPT)])
    plsc.subcore_barrier()

    for k in range(NBUF):
        pltpu.async_copy(table_sh.at[ridx_v.at[k]], bufs[k], gsems[k])

    def body(i, _):
        jb = NBUF * i
        for k in range(NBUF):
            j = jb + k
            pltpu.make_async_copy(table_sh.at[ridx_v.at[j]], bufs[k],
                                  gsems[k]).wait()
            pltpu.async_copy(bufs[k], acc_sh.at[cidx_v.at[j]], ssems[k],
                             add=True)
        for k in range(NBUF):
            j = jb + k
            pltpu.make_async_copy(bufs[k], acc_sh.at[cidx_v.at[j]],
                                  ssems[k]).wait()

            @pl.when(j + NBUF < CPW)
            def _():
                pltpu.async_copy(table_sh.at[ridx_v.at[j + NBUF]], bufs[k],
                                 gsems[k])
        return 0

    lax.fori_loop(0, CPW // NBUF, body, 0)

    plsc.subcore_barrier()
    pltpu.sync_copy(acc_sh.at[pl.ds(base, RPT)],
                    out_hbm.at[cid, pl.ds(base, RPT)])


def _pack_mask():
    rowio = lax.broadcasted_iota(jnp.int32, (NPK, 128), 0)
    laneio = lax.broadcasted_iota(jnp.int32, (NPK, 128), 1)
    nodev = 2 * rowio + laneio // H
    return jnp.where(nodev < N, 1.0, 0.0).astype(jnp.float32)


def _tc1(x2_ref, w1x_ref, deg_ref, hs1_ref, dinv_ref):
    dpack = lax.rsqrt(deg_ref[0] + deg_ref[1] + 1.0)
    h_p = jnp.dot(x2_ref[...], w1x_ref[...],
                  preferred_element_type=jnp.float32)
    hs1_ref[...] = h_p * dpack
    dinv_ref[...] = dpack


def _tc2(acc_ref, hs1_ref, dinv_ref, g_ref, be_ref, b1_ref, w2x_ref,
         hs2_ref):
    dpack = dinv_ref[...]
    maskf = _pack_mask()
    s = ((acc_ref[0] + acc_ref[1] + hs1_ref[...]) * dpack
         + b1_ref[...]) * maskf
    inv_n = jnp.float32(1.0 / N)
    mu_p = jnp.sum(s, axis=0, keepdims=True) * inv_n
    mu64 = mu_p[:, :H] + mu_p[:, H:]
    mu = jnp.concatenate([mu64, mu64], axis=1)
    v_p = jnp.sum(s * s, axis=0, keepdims=True) * inv_n
    v64 = v_p[:, :H] + v_p[:, H:]
    var = jnp.concatenate([v64, v64], axis=1) - mu * mu
    g = g_ref[...] * (s - mu) * lax.rsqrt(var + 1e-5) + be_ref[...]
    g = jnp.maximum(g, 0.0) * maskf
    h2 = jnp.dot(g, w2x_ref[...], preferred_element_type=jnp.float32)
    hs2_ref[...] = h2 * dpack


def _tc3(acc_ref, hs2_ref, dinv_ref, b2_ref, out_ref):
    npk = N // 2
    out_ref[...] = ((acc_ref[0, :npk] + acc_ref[1, :npk]
                     + hs2_ref[0:npk]) * dinv_ref[0:npk] + b2_ref[...])


def kernel(x, edge_index, W1, b1, gamma1, beta1, W2, b2):
    pad = jnp.full((EPAD - E,), N, dtype=jnp.int32)
    r_p = jnp.concatenate([edge_index[0], pad]).reshape(NW, CPW, CHUNK)
    c_p = jnp.concatenate([edge_index[1], pad]).reshape(NW, CPW, CHUNK)
    x2 = jnp.pad(x, ((0, NPAD - N), (0, 0))).reshape(NPK, 2 * F_IN)
    z = jnp.zeros((F_IN, H), jnp.float32)
    w1x = jnp.concatenate(
        [jnp.concatenate([W1, z], axis=1), jnp.concatenate([z, W1], axis=1)],
        axis=0)
    w2p = jnp.pad(W2, ((0, 0), (0, H - C)))
    z2 = jnp.zeros((H, H), jnp.float32)
    w2x = jnp.concatenate(
        [jnp.concatenate([w2p, z2], axis=1),
         jnp.concatenate([z2, w2p], axis=1)], axis=0)
    b1p = jnp.tile(b1, 2).reshape(1, 2 * H)
    b2p = jnp.tile(jnp.pad(b2, (0, H - C)), 2).reshape(1, 2 * H)
    g1p = jnp.tile(gamma1, 2).reshape(1, 2 * H)
    be1p = jnp.tile(beta1, 2).reshape(1, 2 * H)

    degp = _deg_kernel(c_p)

    hs1_p, dinv_p = pl.pallas_call(
        _tc1,
        out_shape=[jax.ShapeDtypeStruct((NPK, 128), jnp.float32),
                   jax.ShapeDtypeStruct((NPK, 128), jnp.float32)],
    )(x2, w1x, degp)

    acc1 = _prop(r_p, c_p, hs1_p.reshape(NPAD, D))
    acc1v = acc1.reshape(NC, NPK, 128)

    hs2_p = pl.pallas_call(
        _tc2,
        out_shape=jax.ShapeDtypeStruct((NPK, 128), jnp.float32),
    )(acc1v, hs1_p, dinv_p, g1p, be1p, b1p, w2x)

    acc2 = _prop(r_p, c_p, hs2_p.reshape(NPAD, D))
    acc2v = acc2.reshape(NC, NPK, 128)

    out_p = pl.pallas_call(
        _tc3,
        out_shape=jax.ShapeDtypeStruct((N // 2, 128), jnp.float32),
    )(acc2v, hs2_p, dinv_p, b2p)

    return out_p.reshape(N, H)[:, :C]
